# baseline (device time: 20932 ns/iter reference)
import jax
import jax.numpy as jnp
from jax import lax
from jax.experimental import pallas as pl
from jax.experimental.pallas import tpu as pltpu

N_DEV = 4


def kernel(x, W1, W2):
    m, _ = x.shape
    n = W2.shape[1]

    def body(x_ref, w1_ref, w2_ref, out_ref, comm_ref, send_sems, recv_sems):
        my_pos = lax.axis_index("i")
        left = (my_pos - 1) % N_DEV
        right = (my_pos + 1) % N_DEV

        h = jnp.maximum(
            jnp.dot(x_ref[:, :], w1_ref[:, :], preferred_element_type=jnp.float32),
            0.0,
        )
        partial = jnp.dot(h, w2_ref[:, :], preferred_element_type=jnp.float32)
        out_ref[:, :] = partial
        comm_ref[0, :, :] = partial

        barrier_sem = pltpu.get_barrier_semaphore()
        for nbr in [left, right]:
            pl.semaphore_signal(
                barrier_sem, inc=1,
                device_id=(nbr,), device_id_type=pl.DeviceIdType.MESH,
            )
        pl.semaphore_wait(barrier_sem, 2)

        for hop in range(N_DEV - 1):
            rdma = pltpu.make_async_remote_copy(
                src_ref=comm_ref.at[hop],
                dst_ref=comm_ref.at[hop + 1],
                send_sem=send_sems.at[hop],
                recv_sem=recv_sems.at[hop],
                device_id=(right,),
                device_id_type=pl.DeviceIdType.MESH,
            )
            rdma.start()
            rdma.wait()
            out_ref[:, :] += comm_ref[hop + 1, :, :]

    return pl.pallas_call(
        body,
        out_shape=jax.ShapeDtypeStruct((m, n), jnp.float32),
        in_specs=[pl.BlockSpec(memory_space=pltpu.VMEM)] * 3,
        out_specs=pl.BlockSpec(memory_space=pltpu.VMEM),
        scratch_shapes=[
            pltpu.VMEM((N_DEV, m, n), jnp.float32),
            pltpu.SemaphoreType.DMA((N_DEV - 1,)),
            pltpu.SemaphoreType.DMA((N_DEV - 1,)),
        ],
        compiler_params=pltpu.CompilerParams(collective_id=0),
    )(x, W1, W2)


# device time: 14191 ns/iter; 1.4750x vs baseline; 1.4750x over previous
import jax
import jax.numpy as jnp
from jax import lax
from jax.experimental import pallas as pl
from jax.experimental.pallas import tpu as pltpu

N_DEV = 4


def kernel(x, W1, W2):
    m, _ = x.shape
    n = W2.shape[1]
    qr = m // N_DEV

    def body(x_ref, w1_ref, w2_ref, out_ref,
             rs_buf, rs_send, rs_recv, ag_send, ag_recv):
        my = lax.axis_index("i")

        h = jnp.maximum(
            jnp.dot(x_ref[:, :], w1_ref[:, :], preferred_element_type=jnp.float32),
            0.0,
        )
        out_ref[:, :] = jnp.dot(h, w2_ref[:, :], preferred_element_type=jnp.float32)

        barrier_sem = pltpu.get_barrier_semaphore()
        for jj in range(1, N_DEV):
            pl.semaphore_signal(
                barrier_sem, inc=1,
                device_id=((my + jj) % N_DEV,),
                device_id_type=pl.DeviceIdType.MESH,
            )
        pl.semaphore_wait(barrier_sem, N_DEV - 1)

        rs_out = []
        for jj in range(1, N_DEV):
            r = (my + jj) % N_DEV
            slot = N_DEV - 1 - jj
            rdma = pltpu.make_async_remote_copy(
                src_ref=out_ref.at[pl.ds(r * qr, qr), :],
                dst_ref=rs_buf.at[slot],
                send_sem=rs_send.at[jj - 1],
                recv_sem=rs_recv.at[slot],
                device_id=(r,),
                device_id_type=pl.DeviceIdType.MESH,
            )
            rdma.start()
            rs_out.append(rdma)

        for j in range(N_DEV - 1):
            recv = pltpu.make_async_remote_copy(
                src_ref=rs_buf.at[j], dst_ref=rs_buf.at[j],
                send_sem=rs_send.at[j], recv_sem=rs_recv.at[j],
                device_id=(my,), device_id_type=pl.DeviceIdType.MESH,
            )
            recv.wait_recv()
        for rdma in rs_out:
            rdma.wait_send()

        mine = pl.ds(my * qr, qr)
        out_ref[mine, :] = (
            out_ref[mine, :] + rs_buf[0] + rs_buf[1] + rs_buf[2]
        )

        ag_out = []
        for jj in range(1, N_DEV):
            r = (my + jj) % N_DEV
            slot = N_DEV - 1 - jj
            rdma = pltpu.make_async_remote_copy(
                src_ref=out_ref.at[mine, :],
                dst_ref=out_ref.at[mine, :],
                send_sem=ag_send.at[jj - 1],
                recv_sem=ag_recv.at[slot],
                device_id=(r,),
                device_id_type=pl.DeviceIdType.MESH,
            )
            rdma.start()
            ag_out.append(rdma)

        for j in range(N_DEV - 1):
            s_rows = pl.ds(((my + 1 + j) % N_DEV) * qr, qr)
            recv = pltpu.make_async_remote_copy(
                src_ref=out_ref.at[s_rows, :], dst_ref=out_ref.at[s_rows, :],
                send_sem=ag_send.at[j], recv_sem=ag_recv.at[j],
                device_id=(my,), device_id_type=pl.DeviceIdType.MESH,
            )
            recv.wait_recv()
        for rdma in ag_out:
            rdma.wait_send()

    return pl.pallas_call(
        body,
        out_shape=jax.ShapeDtypeStruct((m, n), jnp.float32),
        in_specs=[pl.BlockSpec(memory_space=pltpu.VMEM)] * 3,
        out_specs=pl.BlockSpec(memory_space=pltpu.VMEM),
        scratch_shapes=[
            pltpu.VMEM((N_DEV - 1, qr, n), jnp.float32),
            pltpu.SemaphoreType.DMA((N_DEV - 1,)),
            pltpu.SemaphoreType.DMA((N_DEV - 1,)),
            pltpu.SemaphoreType.DMA((N_DEV - 1,)),
            pltpu.SemaphoreType.DMA((N_DEV - 1,)),
        ],
        compiler_params=pltpu.CompilerParams(collective_id=0),
    )(x, W1, W2)


# device time: 12965 ns/iter; 1.6145x vs baseline; 1.0946x over previous
import jax
import jax.numpy as jnp
from jax import lax
from jax.experimental import pallas as pl
from jax.experimental.pallas import tpu as pltpu

N_DEV = 4


def kernel(x, W1, W2):
    m, _ = x.shape
    n = W2.shape[1]
    qr = m // N_DEV

    def body(x_ref, w1_ref, w2_ref, out_ref,
             rs_stage, rs_buf, ag_stage, ag_buf,
             rs_send, rs_recv, ag_send, ag_recv):
        my = lax.axis_index("i")

        h = jnp.maximum(
            jnp.dot(x_ref[:, :], w1_ref[:, :], preferred_element_type=jnp.float32),
            0.0,
        )
        partial = jnp.dot(h, w2_ref[:, :], preferred_element_type=jnp.float32)
        out_ref[:, :] = partial

        barrier_sem = pltpu.get_barrier_semaphore()
        for jj in range(1, N_DEV):
            pl.semaphore_signal(
                barrier_sem, inc=1,
                device_id=((my + jj) % N_DEV,),
                device_id_type=pl.DeviceIdType.MESH,
            )
        pl.semaphore_wait(barrier_sem, N_DEV - 1)

        rs_out = []
        for jj in (2, 1, 3):
            r = (my + jj) % N_DEV
            slot = N_DEV - 1 - jj
            src_slot = jj - 1
            rs_stage[src_slot, :, :] = out_ref[
                pl.ds(r * qr, qr), :
            ].astype(jnp.bfloat16)
            rdma = pltpu.make_async_remote_copy(
                src_ref=rs_stage.at[src_slot],
                dst_ref=rs_buf.at[slot],
                send_sem=rs_send.at[src_slot],
                recv_sem=rs_recv.at[slot],
                device_id=(r,),
                device_id_type=pl.DeviceIdType.MESH,
            )
            rdma.start()
            rs_out.append(rdma)

        for j in range(N_DEV - 1):
            recv = pltpu.make_async_remote_copy(
                src_ref=rs_buf.at[j], dst_ref=rs_buf.at[j],
                send_sem=rs_send.at[j], recv_sem=rs_recv.at[j],
                device_id=(my,), device_id_type=pl.DeviceIdType.MESH,
            )
            recv.wait_recv()

        mine = pl.ds(my * qr, qr)
        reduced = (
            out_ref[mine, :]
            + rs_buf[0].astype(jnp.float32)
            + rs_buf[1].astype(jnp.float32)
            + rs_buf[2].astype(jnp.float32)
        )
        out_ref[mine, :] = reduced
        ag_stage[:, :] = reduced.astype(jnp.bfloat16)

        ag_out = []
        for jj in range(1, N_DEV):
            r = (my + jj) % N_DEV
            slot = N_DEV - 1 - jj
            rdma = pltpu.make_async_remote_copy(
                src_ref=ag_stage,
                dst_ref=ag_buf.at[slot],
                send_sem=ag_send.at[jj - 1],
                recv_sem=ag_recv.at[slot],
                device_id=(r,),
                device_id_type=pl.DeviceIdType.MESH,
            )
            rdma.start()
            ag_out.append(rdma)

        for j in range(N_DEV - 1):
            recv = pltpu.make_async_remote_copy(
                src_ref=ag_buf.at[j], dst_ref=ag_buf.at[j],
                send_sem=ag_send.at[j], recv_sem=ag_recv.at[j],
                device_id=(my,), device_id_type=pl.DeviceIdType.MESH,
            )
            recv.wait_recv()
            s_rows = pl.ds(((my + 1 + j) % N_DEV) * qr, qr)
            out_ref[s_rows, :] = ag_buf[j].astype(jnp.float32)

        for rdma in rs_out:
            rdma.wait_send()
        for rdma in ag_out:
            rdma.wait_send()

    return pl.pallas_call(
        body,
        out_shape=jax.ShapeDtypeStruct((m, n), jnp.float32),
        in_specs=[pl.BlockSpec(memory_space=pltpu.VMEM)] * 3,
        out_specs=pl.BlockSpec(memory_space=pltpu.VMEM),
        scratch_shapes=[
            pltpu.VMEM((N_DEV - 1, qr, n), jnp.bfloat16),
            pltpu.VMEM((N_DEV - 1, qr, n), jnp.bfloat16),
            pltpu.VMEM((qr, n), jnp.bfloat16),
            pltpu.VMEM((N_DEV - 1, qr, n), jnp.bfloat16),
            pltpu.SemaphoreType.DMA((N_DEV - 1,)),
            pltpu.SemaphoreType.DMA((N_DEV - 1,)),
            pltpu.SemaphoreType.DMA((N_DEV - 1,)),
            pltpu.SemaphoreType.DMA((N_DEV - 1,)),
        ],
        compiler_params=pltpu.CompilerParams(collective_id=0),
    )(x, W1, W2)


# device time: 12611 ns/iter; 1.6598x vs baseline; 1.0281x over previous
import jax
import jax.numpy as jnp
from jax import lax
from jax.experimental import pallas as pl
from jax.experimental.pallas import tpu as pltpu

N_DEV = 4
N_HALF = 2


def kernel(x, W1, W2):
    m, _ = x.shape
    n = W2.shape[1]
    qr = m // N_DEV
    hr = qr // N_HALF

    def body(x_ref, w1_ref, w2_ref, out_ref,
             rs_stage, rs_buf, ag_stage, ag_buf,
             rs_send, rs_recv, ag_send, ag_recv):
        my = lax.axis_index("i")

        barrier_sem = pltpu.get_barrier_semaphore()
        for jj in range(1, N_DEV):
            pl.semaphore_signal(
                barrier_sem, inc=1,
                device_id=((my + jj) % N_DEV,),
                device_id_type=pl.DeviceIdType.MESH,
            )

        h = jnp.maximum(
            jnp.dot(x_ref[:, :], w1_ref[:, :], preferred_element_type=jnp.float32),
            0.0,
        )
        out_ref[:, :] = jnp.dot(h, w2_ref[:, :], preferred_element_type=jnp.float32)

        for jj in range(1, N_DEV):
            r = (my + jj) % N_DEV
            rs_stage[jj - 1, :, :] = out_ref[
                pl.ds(r * qr, qr), :
            ].astype(jnp.bfloat16)

        pl.semaphore_wait(barrier_sem, N_DEV - 1)

        rs_out = []
        for half in range(N_HALF):
            rows = pl.ds(half * hr, hr)
            for jj in (2, 1, 3):
                r = (my + jj) % N_DEV
                slot = N_DEV - 1 - jj
                rdma = pltpu.make_async_remote_copy(
                    src_ref=rs_stage.at[jj - 1, rows, :],
                    dst_ref=rs_buf.at[slot, rows, :],
                    send_sem=rs_send.at[half * 3 + jj - 1],
                    recv_sem=rs_recv.at[half * 3 + slot],
                    device_id=(r,),
                    device_id_type=pl.DeviceIdType.MESH,
                )
                rdma.start()
                rs_out.append(rdma)

        ag_out = []
        for half in range(N_HALF):
            rows = pl.ds(half * hr, hr)
            for j in range(N_DEV - 1):
                recv = pltpu.make_async_remote_copy(
                    src_ref=rs_buf.at[j, rows, :],
                    dst_ref=rs_buf.at[j, rows, :],
                    send_sem=rs_send.at[half * 3 + j],
                    recv_sem=rs_recv.at[half * 3 + j],
                    device_id=(my,), device_id_type=pl.DeviceIdType.MESH,
                )
                recv.wait_recv()

            mine = pl.ds(my * qr + half * hr, hr)
            reduced = (
                out_ref[mine, :]
                + rs_buf[0, pl.ds(half * hr, hr), :].astype(jnp.float32)
                + rs_buf[1, pl.ds(half * hr, hr), :].astype(jnp.float32)
                + rs_buf[2, pl.ds(half * hr, hr), :].astype(jnp.float32)
            )
            out_ref[mine, :] = reduced
            ag_stage[rows, :] = reduced.astype(jnp.bfloat16)

            for jj in (2, 1, 3):
                r = (my + jj) % N_DEV
                slot = N_DEV - 1 - jj
                rdma = pltpu.make_async_remote_copy(
                    src_ref=ag_stage.at[rows, :],
                    dst_ref=ag_buf.at[slot, rows, :],
                    send_sem=ag_send.at[half * 3 + jj - 1],
                    recv_sem=ag_recv.at[half * 3 + slot],
                    device_id=(r,),
                    device_id_type=pl.DeviceIdType.MESH,
                )
                rdma.start()
                ag_out.append(rdma)

        for half in range(N_HALF):
            rows = pl.ds(half * hr, hr)
            for j in range(N_DEV - 1):
                recv = pltpu.make_async_remote_copy(
                    src_ref=ag_buf.at[j, rows, :],
                    dst_ref=ag_buf.at[j, rows, :],
                    send_sem=ag_send.at[half * 3 + j],
                    recv_sem=ag_recv.at[half * 3 + j],
                    device_id=(my,), device_id_type=pl.DeviceIdType.MESH,
                )
                recv.wait_recv()
                s_rows = pl.ds(((my + 1 + j) % N_DEV) * qr + half * hr, hr)
                out_ref[s_rows, :] = ag_buf[
                    j, pl.ds(half * hr, hr), :
                ].astype(jnp.float32)

        for rdma in rs_out:
            rdma.wait_send()
        for rdma in ag_out:
            rdma.wait_send()

    return pl.pallas_call(
        body,
        out_shape=jax.ShapeDtypeStruct((m, n), jnp.float32),
        in_specs=[pl.BlockSpec(memory_space=pltpu.VMEM)] * 3,
        out_specs=pl.BlockSpec(memory_space=pltpu.VMEM),
        scratch_shapes=[
            pltpu.VMEM((N_DEV - 1, qr, n), jnp.bfloat16),
            pltpu.VMEM((N_DEV - 1, qr, n), jnp.bfloat16),
            pltpu.VMEM((qr, n), jnp.bfloat16),
            pltpu.VMEM((N_DEV - 1, qr, n), jnp.bfloat16),
            pltpu.SemaphoreType.DMA((N_HALF * 3,)),
            pltpu.SemaphoreType.DMA((N_HALF * 3,)),
            pltpu.SemaphoreType.DMA((N_HALF * 3,)),
            pltpu.SemaphoreType.DMA((N_HALF * 3,)),
        ],
        compiler_params=pltpu.CompilerParams(collective_id=0),
    )(x, W1, W2)


# device time: 11746 ns/iter; 1.7821x vs baseline; 1.0736x over previous
import jax
import jax.numpy as jnp
from jax import lax
from jax.experimental import pallas as pl
from jax.experimental.pallas import tpu as pltpu

N_DEV = 4
N_CHUNK = 4


def kernel(x, W1, W2):
    m, _ = x.shape
    n = W2.shape[1]
    cr = m // N_CHUNK

    def body(x_ref, w1_ref, w2_ref, out_ref, stage, buf, snd, rcv):
        my = lax.axis_index("i")

        barrier_sem = pltpu.get_barrier_semaphore()
        for jj in range(1, N_DEV):
            pl.semaphore_signal(
                barrier_sem, inc=1,
                device_id=((my + jj) % N_DEV,),
                device_id_type=pl.DeviceIdType.MESH,
            )

        partials = []
        sends = []
        for c in range(N_CHUNK):
            rows = slice(c * cr, (c + 1) * cr)
            h = jnp.maximum(
                jnp.dot(x_ref[rows, :], w1_ref[:, :],
                        preferred_element_type=jnp.float32),
                0.0,
            )
            p = jnp.dot(h, w2_ref[:, :], preferred_element_type=jnp.float32)
            partials.append(p)
            stage[rows, :] = p.astype(jnp.bfloat16)
            if c == 0:
                pl.semaphore_wait(barrier_sem, N_DEV - 1)
            for jj in (2, 1, 3):
                r = (my + jj) % N_DEV
                slot = N_DEV - 1 - jj
                rdma = pltpu.make_async_remote_copy(
                    src_ref=stage.at[rows, :],
                    dst_ref=buf.at[slot, rows, :],
                    send_sem=snd.at[c * 3 + jj - 1],
                    recv_sem=rcv.at[c * 3 + slot],
                    device_id=(r,),
                    device_id_type=pl.DeviceIdType.MESH,
                )
                rdma.start()
                sends.append(rdma)

        for c in range(N_CHUNK):
            rows = slice(c * cr, (c + 1) * cr)
            for j in range(N_DEV - 1):
                recv = pltpu.make_async_remote_copy(
                    src_ref=buf.at[j, rows, :],
                    dst_ref=buf.at[j, rows, :],
                    send_sem=snd.at[c * 3 + j],
                    recv_sem=rcv.at[c * 3 + j],
                    device_id=(my,), device_id_type=pl.DeviceIdType.MESH,
                )
                recv.wait_recv()
            out_ref[rows, :] = (
                partials[c]
                + buf[0, rows, :].astype(jnp.float32)
                + buf[1, rows, :].astype(jnp.float32)
                + buf[2, rows, :].astype(jnp.float32)
            )

        for rdma in sends:
            rdma.wait_send()

    return pl.pallas_call(
        body,
        out_shape=jax.ShapeDtypeStruct((m, n), jnp.float32),
        in_specs=[pl.BlockSpec(memory_space=pltpu.VMEM)] * 3,
        out_specs=pl.BlockSpec(memory_space=pltpu.VMEM),
        scratch_shapes=[
            pltpu.VMEM((m, n), jnp.bfloat16),
            pltpu.VMEM((N_DEV - 1, m, n), jnp.bfloat16),
            pltpu.SemaphoreType.DMA((N_CHUNK * 3,)),
            pltpu.SemaphoreType.DMA((N_CHUNK * 3,)),
        ],
        compiler_params=pltpu.CompilerParams(collective_id=0),
    )(x, W1, W2)
